# baseline (speedup 1.0000x reference)
import functools

import jax
import jax.numpy as jnp
from jax import lax
from jax.experimental import pallas as pl
from jax.experimental.pallas import tpu as pltpu
from jax.experimental.pallas import tpu_sc as plsc

N = 10000
E = 320000
D = 128
HOP = 10

NC = 2
NS = 16
NW = NC * NS
CHUNK = 128
NCH = 80
EPW = NCH * CHUNK
EP = NW * EPW
NPAD = 10112
ACC_TILE_ROWS = NPAD // NS
LAST_TILE = NS - 1
LAST_ROWS = N - LAST_TILE * ACC_TILE_ROWS

@functools.cache
def _mesh():
    return plsc.VectorSubcoreMesh(core_axis_name="c", subcore_axis_name="s",
                                  num_cores=NC, num_subcores=NS)


def _hop_body(src_hbm, dst_hbm, h_hbm, zeros_hbm, out_hbm,
              src_v, dst_v, rows_v, acc, sem):
    c = lax.axis_index("c")
    t = lax.axis_index("s")
    w = t * NC + c
    lo = pl.multiple_of(t * ACC_TILE_ROWS, 8)
    pltpu.sync_copy(zeros_hbm.at[pl.ds(0, ACC_TILE_ROWS)],
                    acc.at[pl.ds(lo, ACC_TILE_ROWS)])
    pltpu.sync_copy(src_hbm.at[w], src_v)
    pltpu.sync_copy(dst_hbm.at[w], dst_v)
    plsc.subcore_barrier()

    def body(j, carry):
        pltpu.async_copy(h_hbm.at[src_v.at[j]], rows_v, sem).wait()
        pltpu.sync_copy(rows_v, acc.at[dst_v.at[j]], add=True)
        return carry

    lax.fori_loop(0, NCH, body, 0, unroll=False)
    plsc.subcore_barrier()

    @pl.when(t < LAST_TILE)
    def _():
        pltpu.sync_copy(acc.at[pl.ds(lo, ACC_TILE_ROWS)],
                        out_hbm.at[c].at[pl.ds(lo, ACC_TILE_ROWS)])

    @pl.when(t == LAST_TILE)
    def _():
        lo2 = pl.multiple_of(LAST_TILE * ACC_TILE_ROWS, 8)
        pltpu.sync_copy(acc.at[pl.ds(lo2, LAST_ROWS)],
                        out_hbm.at[c].at[pl.ds(lo2, LAST_ROWS)])


@functools.cache
def _hop_call():
    return pl.kernel(
        _hop_body,
        out_type=jax.ShapeDtypeStruct((NC, N, D), jnp.float32),
        mesh=_mesh(),
        scratch_types=[
            pltpu.VMEM((NCH, CHUNK), jnp.int32),
            pltpu.VMEM((NCH, CHUNK), jnp.int32),
            pltpu.VMEM((CHUNK, D), jnp.float32),
            pltpu.VMEM_SHARED((NPAD, D), jnp.float32),
            pltpu.SemaphoreType.DMA,
        ],
    )


def _deg_body(dst_hbm, ones_hbm, zeros_hbm, out_hbm, dst_v, rows_v, acc, sem):
    del sem
    c = lax.axis_index("c")
    t = lax.axis_index("s")
    w = t * NC + c
    lo = pl.multiple_of(t * ACC_TILE_ROWS, 8)
    pltpu.sync_copy(zeros_hbm.at[pl.ds(0, ACC_TILE_ROWS)],
                    acc.at[pl.ds(lo, ACC_TILE_ROWS)])
    pltpu.sync_copy(dst_hbm.at[w], dst_v)
    pltpu.sync_copy(ones_hbm, rows_v)
    plsc.subcore_barrier()

    def body(j, carry):
        pltpu.sync_copy(rows_v, acc.at[dst_v.at[j]], add=True)
        return carry

    lax.fori_loop(0, NCH, body, 0, unroll=False)
    plsc.subcore_barrier()

    @pl.when(t < LAST_TILE)
    def _():
        pltpu.sync_copy(acc.at[pl.ds(lo, ACC_TILE_ROWS)],
                        out_hbm.at[c].at[pl.ds(lo, ACC_TILE_ROWS)])

    @pl.when(t == LAST_TILE)
    def _():
        lo2 = pl.multiple_of(LAST_TILE * ACC_TILE_ROWS, 8)
        pltpu.sync_copy(acc.at[pl.ds(lo2, LAST_ROWS)],
                        out_hbm.at[c].at[pl.ds(lo2, LAST_ROWS)])


@functools.cache
def _deg_call():
    return pl.kernel(
        _deg_body,
        out_type=jax.ShapeDtypeStruct((NC, N, D), jnp.float32),
        mesh=_mesh(),
        scratch_types=[
            pltpu.VMEM((NCH, CHUNK), jnp.int32),
            pltpu.VMEM((CHUNK, D), jnp.float32),
            pltpu.VMEM_SHARED((NPAD, D), jnp.float32),
            pltpu.SemaphoreType.DMA,
        ],
    )


_BLK = 1000


def _mlp_body(x_ref, w1_ref, b1_ref, w2_ref, b2_ref, o_ref):
    h = lax.dot_general(x_ref[...], w1_ref[...], (((1,), (1,)), ((), ())),
                        preferred_element_type=jnp.float32)
    h = jnp.maximum(h + b1_ref[...], 0.0)
    o_ref[...] = lax.dot_general(h, w2_ref[...], (((1,), (1,)), ((), ())),
                                 preferred_element_type=jnp.float32) + b2_ref[...]


def _mlp_call(x, W1, b1r, W2, b2r):
    return pl.pallas_call(
        _mlp_body,
        grid=(N // _BLK,),
        in_specs=[
            pl.BlockSpec((_BLK, D), lambda i: (i, 0)),
            pl.BlockSpec((D, D), lambda i: (0, 0)),
            pl.BlockSpec((1, D), lambda i: (0, 0)),
            pl.BlockSpec((D, D), lambda i: (0, 0)),
            pl.BlockSpec((1, D), lambda i: (0, 0)),
        ],
        out_specs=pl.BlockSpec((_BLK, D), lambda i: (i, 0)),
        out_shape=jax.ShapeDtypeStruct((N, D), jnp.float32),
    )(x, W1, b1r, W2, b2r)


def _inv_body(dp_ref, inv_ref):
    deg = dp_ref[0, :, 0] + dp_ref[1, :, 0]
    inv_ref[...] = (1.0 / jnp.clip(deg, 1.0, None))[:, None]


def _inv_call(degp):
    return pl.pallas_call(
        _inv_body,
        grid=(N // _BLK,),
        in_specs=[pl.BlockSpec((NC, _BLK, D), lambda i: (0, i, 0))],
        out_specs=pl.BlockSpec((_BLK, 1), lambda i: (i, 0)),
        out_shape=jax.ShapeDtypeStruct((N, 1), jnp.float32),
    )(degp)


def _merge_body(p_ref, inv_ref, o_ref):
    o_ref[...] = (p_ref[0] + p_ref[1]) * inv_ref[...]


def _merge_call(p, inv):
    return pl.pallas_call(
        _merge_body,
        grid=(N // _BLK,),
        in_specs=[
            pl.BlockSpec((NC, _BLK, D), lambda i: (0, i, 0)),
            pl.BlockSpec((_BLK, 1), lambda i: (i, 0)),
        ],
        out_specs=pl.BlockSpec((_BLK, D), lambda i: (i, 0)),
        out_shape=jax.ShapeDtypeStruct((N, D), jnp.float32),
    )(p, inv)


def _combine_body(s_ref, *refs):
    h_refs, o_ref = refs[:-1], refs[-1]
    srow = s_ref[...]
    acc = jnp.zeros((_BLK, D), jnp.float32)
    for h_ref in h_refs:
        hk = h_ref[...]
        gate = jax.nn.sigmoid(jnp.sum(hk * srow, axis=1, keepdims=True))
        acc = acc + gate * hk
    o_ref[...] = acc


def _combine_call(srow, hs):
    return pl.pallas_call(
        _combine_body,
        grid=(N // _BLK,),
        in_specs=[pl.BlockSpec((1, D), lambda i: (0, 0))] +
                 [pl.BlockSpec((_BLK, D), lambda i: (i, 0)) for _ in hs],
        out_specs=pl.BlockSpec((_BLK, D), lambda i: (i, 0)),
        out_shape=jax.ShapeDtypeStruct((N, D), jnp.float32),
    )(srow, *hs)


@jax.jit
def kernel(x, edge_index, W1, b1, W2, b2, s):
    src = edge_index[0]
    dst = edge_index[1]
    pad = EP - E
    srcp = jnp.concatenate([src, jnp.zeros((pad,), jnp.int32)]).reshape(
        NW, NCH, CHUNK)
    dstp = jnp.concatenate([dst, jnp.full((pad,), N, jnp.int32)]).reshape(
        NW, NCH, CHUNK)
    zeros2 = jnp.zeros((ACC_TILE_ROWS, D), jnp.float32)
    ones2 = jnp.ones((CHUNK, D), jnp.float32)

    h = _mlp_call(x, W1, b1.reshape(1, D), W2, b2.reshape(1, D))
    inv = _inv_call(_deg_call()(dstp, ones2, zeros2))

    outs = [h]
    for _ in range(HOP):
        p = _hop_call()(srcp, dstp, h, zeros2)
        h = _merge_call(p, inv)
        outs.append(h)

    return _combine_call(s.reshape(1, D), outs)


# speedup vs baseline: 2.7902x; 2.7902x over previous
import functools

import jax
import jax.numpy as jnp
from jax import lax
from jax.experimental import pallas as pl
from jax.experimental.pallas import tpu as pltpu
from jax.experimental.pallas import tpu_sc as plsc

N = 10000
E = 320000
D = 128
HOP = 10

NC = 2
NS = 16
NW = NC * NS
CHUNK = 128
NCH = 80
EPW = NCH * CHUNK
EP = NW * EPW
NPAD = 10112
ACC_TILE_ROWS = NPAD // NS
LAST_TILE = NS - 1
LAST_ROWS = N - LAST_TILE * ACC_TILE_ROWS

@functools.cache
def _mesh():
    return plsc.VectorSubcoreMesh(core_axis_name="c", subcore_axis_name="s",
                                  num_cores=NC, num_subcores=NS)


def _hop_body(src_hbm, dst_hbm, h_hbm, zeros_hbm, out_hbm,
              src_v, dst_v, rows_v, acc, sem):
    c = lax.axis_index("c")
    t = lax.axis_index("s")
    w = t * NC + c
    lo = pl.multiple_of(t * ACC_TILE_ROWS, 8)
    pltpu.sync_copy(zeros_hbm.at[pl.ds(0, ACC_TILE_ROWS)],
                    acc.at[pl.ds(lo, ACC_TILE_ROWS)])
    pltpu.sync_copy(src_hbm.at[w], src_v)
    pltpu.sync_copy(dst_hbm.at[w], dst_v)
    plsc.subcore_barrier()

    def body(j, carry):
        pltpu.async_copy(h_hbm.at[src_v.at[j]], rows_v, sem).wait()
        pltpu.sync_copy(rows_v, acc.at[dst_v.at[j]], add=True)
        return carry

    lax.fori_loop(0, NCH, body, 0, unroll=False)
    plsc.subcore_barrier()

    @pl.when(t < LAST_TILE)
    def _():
        pltpu.sync_copy(acc.at[pl.ds(lo, ACC_TILE_ROWS)],
                        out_hbm.at[c].at[pl.ds(lo, ACC_TILE_ROWS)])

    @pl.when(t == LAST_TILE)
    def _():
        lo2 = pl.multiple_of(LAST_TILE * ACC_TILE_ROWS, 8)
        pltpu.sync_copy(acc.at[pl.ds(lo2, LAST_ROWS)],
                        out_hbm.at[c].at[pl.ds(lo2, LAST_ROWS)])


@functools.cache
def _hop_call():
    return pl.kernel(
        _hop_body,
        out_type=jax.ShapeDtypeStruct((NC, N, D), jnp.float32),
        mesh=_mesh(),
        scratch_types=[
            pltpu.VMEM((NCH, CHUNK), jnp.int32),
            pltpu.VMEM((NCH, CHUNK), jnp.int32),
            pltpu.VMEM((CHUNK, D), jnp.float32),
            pltpu.VMEM_SHARED((NPAD, D), jnp.float32),
            pltpu.SemaphoreType.DMA,
        ],
    )


def _deg_body(dst_hbm, ones_hbm, zeros_hbm, out_hbm, dst_v, rows_v, acc, sem):
    del sem
    c = lax.axis_index("c")
    t = lax.axis_index("s")
    w = t * NC + c
    lo = pl.multiple_of(t * ACC_TILE_ROWS, 8)
    pltpu.sync_copy(zeros_hbm.at[pl.ds(0, ACC_TILE_ROWS)],
                    acc.at[pl.ds(lo, ACC_TILE_ROWS)])
    pltpu.sync_copy(dst_hbm.at[w], dst_v)
    pltpu.sync_copy(ones_hbm, rows_v)
    plsc.subcore_barrier()

    def body(j, carry):
        pltpu.sync_copy(rows_v, acc.at[dst_v.at[j]], add=True)
        return carry

    lax.fori_loop(0, NCH, body, 0, unroll=False)
    plsc.subcore_barrier()

    @pl.when(t < LAST_TILE)
    def _():
        pltpu.sync_copy(acc.at[pl.ds(lo, ACC_TILE_ROWS)],
                        out_hbm.at[c].at[pl.ds(lo, ACC_TILE_ROWS)])

    @pl.when(t == LAST_TILE)
    def _():
        lo2 = pl.multiple_of(LAST_TILE * ACC_TILE_ROWS, 8)
        pltpu.sync_copy(acc.at[pl.ds(lo2, LAST_ROWS)],
                        out_hbm.at[c].at[pl.ds(lo2, LAST_ROWS)])


@functools.cache
def _deg_call():
    return pl.kernel(
        _deg_body,
        out_type=jax.ShapeDtypeStruct((NC, N, D), jnp.float32),
        mesh=_mesh(),
        scratch_types=[
            pltpu.VMEM((NCH, CHUNK), jnp.int32),
            pltpu.VMEM((CHUNK, D), jnp.float32),
            pltpu.VMEM_SHARED((NPAD, D), jnp.float32),
            pltpu.SemaphoreType.DMA,
        ],
    )


_BLK = 1000


def _mlp_body(x_ref, w1_ref, b1_ref, w2_ref, b2_ref, o_ref):
    h = lax.dot_general(x_ref[...], w1_ref[...], (((1,), (1,)), ((), ())),
                        preferred_element_type=jnp.float32)
    h = jnp.maximum(h + b1_ref[...], 0.0)
    o_ref[...] = lax.dot_general(h, w2_ref[...], (((1,), (1,)), ((), ())),
                                 preferred_element_type=jnp.float32) + b2_ref[...]


def _mlp_call(x, W1, b1r, W2, b2r):
    return pl.pallas_call(
        _mlp_body,
        grid=(N // _BLK,),
        in_specs=[
            pl.BlockSpec((_BLK, D), lambda i: (i, 0)),
            pl.BlockSpec((D, D), lambda i: (0, 0)),
            pl.BlockSpec((1, D), lambda i: (0, 0)),
            pl.BlockSpec((D, D), lambda i: (0, 0)),
            pl.BlockSpec((1, D), lambda i: (0, 0)),
        ],
        out_specs=pl.BlockSpec((_BLK, D), lambda i: (i, 0)),
        out_shape=jax.ShapeDtypeStruct((N, D), jnp.float32),
    )(x, W1, b1r, W2, b2r)


def _inv_body(dp_ref, inv_ref):
    deg = dp_ref[0, :, 0] + dp_ref[1, :, 0]
    inv_ref[...] = (1.0 / jnp.clip(deg, 1.0, None))[:, None]


def _inv_call(degp):
    return pl.pallas_call(
        _inv_body,
        grid=(N // _BLK,),
        in_specs=[pl.BlockSpec((NC, _BLK, D), lambda i: (0, i, 0))],
        out_specs=pl.BlockSpec((_BLK, 1), lambda i: (i, 0)),
        out_shape=jax.ShapeDtypeStruct((N, 1), jnp.float32),
    )(degp)


def _merge_body(p_ref, inv_ref, o_ref):
    o_ref[...] = (p_ref[0] + p_ref[1]) * inv_ref[...]


def _merge_call(p, inv):
    return pl.pallas_call(
        _merge_body,
        grid=(N // _BLK,),
        in_specs=[
            pl.BlockSpec((NC, _BLK, D), lambda i: (0, i, 0)),
            pl.BlockSpec((_BLK, 1), lambda i: (i, 0)),
        ],
        out_specs=pl.BlockSpec((_BLK, D), lambda i: (i, 0)),
        out_shape=jax.ShapeDtypeStruct((N, D), jnp.float32),
    )(p, inv)


def _combine_body(s_ref, *refs):
    h_refs, o_ref = refs[:-1], refs[-1]
    srow = s_ref[...]
    acc = jnp.zeros((_BLK, D), jnp.float32)
    for h_ref in h_refs:
        hk = h_ref[...]
        gate = jax.nn.sigmoid(jnp.sum(hk * srow, axis=1, keepdims=True))
        acc = acc + gate * hk
    o_ref[...] = acc


def _combine_call(srow, hs):
    return pl.pallas_call(
        _combine_body,
        grid=(N // _BLK,),
        in_specs=[pl.BlockSpec((1, D), lambda i: (0, 0))] +
                 [pl.BlockSpec((_BLK, D), lambda i: (i, 0)) for _ in hs],
        out_specs=pl.BlockSpec((_BLK, D), lambda i: (i, 0)),
        out_shape=jax.ShapeDtypeStruct((N, D), jnp.float32),
    )(srow, *hs)


@jax.jit
def kernel(x, edge_index, W1, b1, W2, b2, s):
    src = edge_index[0]
    dst = edge_index[1]
    pad = EP - E
    pad_src = (jnp.arange(pad, dtype=jnp.int32) * 37) % N
    pad_dst = N + (jnp.arange(pad, dtype=jnp.int32) % (NPAD - N))
    srcp = jnp.concatenate([src, pad_src]).reshape(NW, NCH, CHUNK)
    dstp = jnp.concatenate([dst, pad_dst]).reshape(NW, NCH, CHUNK)
    zeros2 = jnp.zeros((ACC_TILE_ROWS, D), jnp.float32)
    ones2 = jnp.ones((CHUNK, D), jnp.float32)

    h = _mlp_call(x, W1, b1.reshape(1, D), W2, b2.reshape(1, D))
    inv = _inv_call(_deg_call()(dstp, ones2, zeros2))

    outs = [h]
    for _ in range(HOP):
        p = _hop_call()(srcp, dstp, h, zeros2)
        h = _merge_call(p, inv)
        outs.append(h)

    return _combine_call(s.reshape(1, D), outs)


# speedup vs baseline: 4.1481x; 1.4867x over previous
import functools

import jax
import jax.numpy as jnp
from jax import lax
from jax.experimental import pallas as pl
from jax.experimental.pallas import tpu as pltpu
from jax.experimental.pallas import tpu_sc as plsc

N = 10000
E = 320000
D = 128
HOP = 10

NC = 2
NS = 16
NW = NC * NS
CHUNK = 128
NCH = 80
NBUF = 2
SEG = 2
SEG_CH = NCH // SEG
EPW = NCH * CHUNK
EP = NW * EPW
NPAD = 10112
ACC_TILE_ROWS = NPAD // NS
LAST_TILE = NS - 1
LAST_ROWS = N - LAST_TILE * ACC_TILE_ROWS

@functools.cache
def _mesh():
    return plsc.VectorSubcoreMesh(core_axis_name="c", subcore_axis_name="s",
                                  num_cores=NC, num_subcores=NS)


def _hop_body(src_hbm, dst_hbm, h_hbm, zeros_hbm, out_hbm,
              src_v, dst_v, rows_bufs, sems, acc):
    c = lax.axis_index("c")
    t = lax.axis_index("s")
    w = t * NC + c
    lo = pl.multiple_of(t * ACC_TILE_ROWS, 8)
    pltpu.sync_copy(zeros_hbm.at[pl.ds(0, ACC_TILE_ROWS)],
                    acc.at[pl.ds(lo, ACC_TILE_ROWS)])
    plsc.subcore_barrier()

    for seg in range(SEG):
        pltpu.sync_copy(src_hbm.at[w, seg], src_v)
        pltpu.sync_copy(dst_hbm.at[w, seg], dst_v)
        for k in range(NBUF):
            pltpu.async_copy(h_hbm.at[src_v.at[k]], rows_bufs[k], sems[k])

        def body(i, carry):
            j = i * NBUF
            for k in range(NBUF):
                pltpu.make_async_copy(h_hbm.at[src_v.at[j + k]],
                                      rows_bufs[k], sems[k]).wait()
                pltpu.sync_copy(rows_bufs[k], acc.at[dst_v.at[j + k]], add=True)

                @pl.when(j + k + NBUF < SEG_CH)
                def _():
                    pltpu.async_copy(h_hbm.at[src_v.at[j + k + NBUF]],
                                     rows_bufs[k], sems[k])
            return carry

        lax.fori_loop(0, SEG_CH // NBUF, body, 0, unroll=False)
    plsc.subcore_barrier()

    @pl.when(t < LAST_TILE)
    def _():
        pltpu.sync_copy(acc.at[pl.ds(lo, ACC_TILE_ROWS)],
                        out_hbm.at[c].at[pl.ds(lo, ACC_TILE_ROWS)])

    @pl.when(t == LAST_TILE)
    def _():
        lo2 = pl.multiple_of(LAST_TILE * ACC_TILE_ROWS, 8)
        pltpu.sync_copy(acc.at[pl.ds(lo2, LAST_ROWS)],
                        out_hbm.at[c].at[pl.ds(lo2, LAST_ROWS)])


@functools.cache
def _hop_call():
    return pl.kernel(
        _hop_body,
        out_type=jax.ShapeDtypeStruct((NC, N, D), jnp.float32),
        mesh=_mesh(),
        scratch_types=[
            pltpu.VMEM((SEG_CH, CHUNK), jnp.int32),
            pltpu.VMEM((SEG_CH, CHUNK), jnp.int32),
            [pltpu.VMEM((CHUNK, D), jnp.float32) for _ in range(NBUF)],
            [pltpu.SemaphoreType.DMA for _ in range(NBUF)],
            pltpu.VMEM_SHARED((NPAD, D), jnp.float32),
        ],
    )


def _deg_body(dst_hbm, ones_hbm, zeros_hbm, out_hbm, dst_v, rows_v, acc, sem):
    del sem
    c = lax.axis_index("c")
    t = lax.axis_index("s")
    w = t * NC + c
    lo = pl.multiple_of(t * ACC_TILE_ROWS, 8)
    pltpu.sync_copy(zeros_hbm.at[pl.ds(0, ACC_TILE_ROWS)],
                    acc.at[pl.ds(lo, ACC_TILE_ROWS)])
    pltpu.sync_copy(dst_hbm.at[w], dst_v)
    pltpu.sync_copy(ones_hbm, rows_v)
    plsc.subcore_barrier()

    for seg in range(SEG):
        def body(j, carry):
            pltpu.sync_copy(rows_v, acc.at[dst_v.at[seg, j]], add=True)
            return carry

        lax.fori_loop(0, SEG_CH, body, 0, unroll=False)
    plsc.subcore_barrier()

    @pl.when(t < LAST_TILE)
    def _():
        pltpu.sync_copy(acc.at[pl.ds(lo, ACC_TILE_ROWS)],
                        out_hbm.at[c].at[pl.ds(lo, ACC_TILE_ROWS)])

    @pl.when(t == LAST_TILE)
    def _():
        lo2 = pl.multiple_of(LAST_TILE * ACC_TILE_ROWS, 8)
        pltpu.sync_copy(acc.at[pl.ds(lo2, LAST_ROWS)],
                        out_hbm.at[c].at[pl.ds(lo2, LAST_ROWS)])


@functools.cache
def _deg_call():
    return pl.kernel(
        _deg_body,
        out_type=jax.ShapeDtypeStruct((NC, N, D), jnp.float32),
        mesh=_mesh(),
        scratch_types=[
            pltpu.VMEM((SEG, SEG_CH, CHUNK), jnp.int32),
            pltpu.VMEM((CHUNK, D), jnp.float32),
            pltpu.VMEM_SHARED((NPAD, D), jnp.float32),
            pltpu.SemaphoreType.DMA,
        ],
    )


_BLK = 1000


def _mlp_body(x_ref, w1_ref, b1_ref, w2_ref, b2_ref, o_ref):
    h = lax.dot_general(x_ref[...], w1_ref[...], (((1,), (1,)), ((), ())),
                        preferred_element_type=jnp.float32)
    h = jnp.maximum(h + b1_ref[...], 0.0)
    o_ref[...] = lax.dot_general(h, w2_ref[...], (((1,), (1,)), ((), ())),
                                 preferred_element_type=jnp.float32) + b2_ref[...]


def _mlp_call(x, W1, b1r, W2, b2r):
    return pl.pallas_call(
        _mlp_body,
        grid=(N // _BLK,),
        in_specs=[
            pl.BlockSpec((_BLK, D), lambda i: (i, 0)),
            pl.BlockSpec((D, D), lambda i: (0, 0)),
            pl.BlockSpec((1, D), lambda i: (0, 0)),
            pl.BlockSpec((D, D), lambda i: (0, 0)),
            pl.BlockSpec((1, D), lambda i: (0, 0)),
        ],
        out_specs=pl.BlockSpec((_BLK, D), lambda i: (i, 0)),
        out_shape=jax.ShapeDtypeStruct((N, D), jnp.float32),
    )(x, W1, b1r, W2, b2r)


def _inv_body(dp_ref, inv_ref):
    deg = dp_ref[0, :, 0] + dp_ref[1, :, 0]
    inv_ref[...] = (1.0 / jnp.clip(deg, 1.0, None))[:, None]


def _inv_call(degp):
    return pl.pallas_call(
        _inv_body,
        grid=(N // _BLK,),
        in_specs=[pl.BlockSpec((NC, _BLK, D), lambda i: (0, i, 0))],
        out_specs=pl.BlockSpec((_BLK, 1), lambda i: (i, 0)),
        out_shape=jax.ShapeDtypeStruct((N, 1), jnp.float32),
    )(degp)


def _merge_body(p_ref, inv_ref, o_ref):
    o_ref[...] = (p_ref[0] + p_ref[1]) * inv_ref[...]


def _merge_call(p, inv):
    return pl.pallas_call(
        _merge_body,
        grid=(N // _BLK,),
        in_specs=[
            pl.BlockSpec((NC, _BLK, D), lambda i: (0, i, 0)),
            pl.BlockSpec((_BLK, 1), lambda i: (i, 0)),
        ],
        out_specs=pl.BlockSpec((_BLK, D), lambda i: (i, 0)),
        out_shape=jax.ShapeDtypeStruct((N, D), jnp.float32),
    )(p, inv)


def _combine_body(s_ref, *refs):
    h_refs, o_ref = refs[:-1], refs[-1]
    srow = s_ref[...]
    acc = jnp.zeros((_BLK, D), jnp.float32)
    for h_ref in h_refs:
        hk = h_ref[...]
        gate = jax.nn.sigmoid(jnp.sum(hk * srow, axis=1, keepdims=True))
        acc = acc + gate * hk
    o_ref[...] = acc


def _combine_call(srow, hs):
    return pl.pallas_call(
        _combine_body,
        grid=(N // _BLK,),
        in_specs=[pl.BlockSpec((1, D), lambda i: (0, 0))] +
                 [pl.BlockSpec((_BLK, D), lambda i: (i, 0)) for _ in hs],
        out_specs=pl.BlockSpec((_BLK, D), lambda i: (i, 0)),
        out_shape=jax.ShapeDtypeStruct((N, D), jnp.float32),
    )(srow, *hs)


@jax.jit
def kernel(x, edge_index, W1, b1, W2, b2, s):
    src = edge_index[0]
    dst = edge_index[1]
    pad = EP - E
    pad_src = (jnp.arange(pad, dtype=jnp.int32) * 37) % N
    pad_dst = N + (jnp.arange(pad, dtype=jnp.int32) % (NPAD - N))
    srcp = jnp.concatenate([src, pad_src]).reshape(NW, SEG, SEG_CH, CHUNK)
    dstp = jnp.concatenate([dst, pad_dst]).reshape(NW, SEG, SEG_CH, CHUNK)
    zeros2 = jnp.zeros((ACC_TILE_ROWS, D), jnp.float32)
    ones2 = jnp.ones((CHUNK, D), jnp.float32)

    h = _mlp_call(x, W1, b1.reshape(1, D), W2, b2.reshape(1, D))
    inv = _inv_call(_deg_call()(dstp, ones2, zeros2))

    outs = [h]
    for _ in range(HOP):
        p = _hop_call()(srcp, dstp, h, zeros2)
        h = _merge_call(p, inv)
        outs.append(h)

    return _combine_call(s.reshape(1, D), outs)
